# baseline (speedup 1.0000x reference)
import jax
import jax.numpy as jnp
from jax import lax
from jax.experimental import pallas as pl
from jax.experimental.pallas import tpu as pltpu
from jax.experimental.pallas import tpu_sc as plsc

_B, _D, _H, _W = 8, 256, 32, 32
_M = 1024
_HW = _H * _W
_N = _B * _HW
_T = 256
_NT = _N // _T
_TPB = _HW // _T
_NEG = -1e30
_NC, _NS = 2, 16
_NW = _NC * _NS
_RPW = _N // _NW


def _norm_rows(q):
    n2 = jnp.sum(q * q, axis=1, keepdims=True)
    return q * lax.rsqrt(jnp.maximum(n2, 1e-24))


def _score_of(qn, keys):
    return lax.dot_general(qn, keys, (((1,), (1,)), ((), ())),
                           preferred_element_type=jnp.float32)


def _split_hi_lo(x):
    hi = x.astype(jnp.bfloat16).astype(jnp.float32)
    return hi, x - hi


def _stats_kernel(qf_ref, keys_ref, cs_ref, cme_ref, kn2_ref):
    i = pl.program_id(0)
    keys = keys_ref[...]
    escore = jnp.exp(_score_of(_norm_rows(qf_ref[...]), keys))

    @pl.when(i == 0)
    def _():
        cs_ref[...] = jnp.zeros((1, _M), jnp.float32)
        cme_ref[...] = jnp.zeros((1, _M), jnp.float32)
        ones_row = jnp.ones((1, _D), jnp.float32)
        kn2_ref[...] = lax.dot_general(ones_row, keys * keys,
                                       (((1,), (1,)), ((), ())),
                                       preferred_element_type=jnp.float32)

    cs_ref[...] += jnp.sum(escore, axis=0, keepdims=True)
    cme_ref[...] = jnp.maximum(cme_ref[...], jnp.max(escore, axis=0, keepdims=True))


def _main_kernel(qf_ref, keys_ref, cs_ref, cme_ref, kn2_ref,
                 sm_ref, sq_ref, qcat_ref, g_ref, s_ref, gi_ref, wq_ref,
                 gscr, sscr):
    i = pl.program_id(0)
    keys = keys_ref[...]
    qn = _norm_rows(qf_ref[...])
    qcat_ref[:, :_D] = qn
    score = _score_of(qn, keys)

    rmax = jnp.max(score, axis=1, keepdims=True)
    e = jnp.exp(score - rmax)
    rsum = jnp.sum(e, axis=1, keepdims=True)
    sm = e * (1.0 / rsum)
    sm_ref[...] = sm

    u = jnp.exp(rmax)
    v = 1.0 / cs_ref[...]
    sq_ref[...] = e * u * v

    qcat_ref[:, _D:] = lax.dot_general(sm, keys, (((1,), (0,)), ((), ())),
                                       preferred_element_type=jnp.float32)

    b1 = score >= rmax
    oh1 = b1.astype(jnp.float32)
    masked = jnp.where(b1, _NEG, score)
    m2 = jnp.max(masked, axis=1, keepdims=True)
    oh2 = (masked >= m2).astype(jnp.float32)

    cme_hi, cme_lo = _split_hi_lo(cme_ref[...])
    io_hi, io_lo = _split_hi_lo(
        lax.broadcasted_iota(jnp.int32, (1, _M), 1).astype(jnp.float32))
    vrows = jnp.concatenate(
        [cme_hi, cme_lo, kn2_ref[...], io_hi, io_lo], axis=0)
    g1 = lax.dot_general(oh1, vrows, (((1,), (1,)), ((), ())),
                         preferred_element_type=jnp.float32)
    cme_g = g1[:, 0:1] + g1[:, 1:2]
    kn2_g = g1[:, 2:3]
    gi_f = g1[:, 3:4] + g1[:, 4:5]
    gi_row = jnp.transpose(jnp.clip(gi_f + 0.5, 0.0, _M - 1.0))
    gi_ref[pl.ds(i, 1), :] = gi_row.astype(jnp.int32)
    kn2_g2 = lax.dot_general(oh2, kn2_ref[...], (((1,), (1,)), ((), ())),
                             preferred_element_type=jnp.float32)

    @pl.when(i == 0)
    def _():
        gscr[...] = jnp.zeros((_B, 1), jnp.float32)
        sscr[...] = jnp.zeros((_B, 1), jnp.float32)

    boh = lax.broadcasted_iota(jnp.int32, (_B, 1), 0) == (i // _TPB)

    d1sq = 1.0 - 2.0 * rmax + kn2_g
    gscr[...] += jnp.where(boh, jnp.sum(d1sq) / (_HW * _D * 1.0), 0.0)

    d2sq = 1.0 - 2.0 * m2 + kn2_g2
    dp = jnp.sqrt(jnp.maximum(d1sq, 0.0))
    dn = jnp.sqrt(jnp.maximum(d2sq, 0.0))
    s_row = jnp.maximum(dp - dn + 1.0, 0.0)
    sscr[...] += jnp.where(boh, jnp.sum(s_row) / (_HW * 1.0), 0.0)

    wgt = u * (1.0 / cme_g)
    wq_ref[...] = jnp.transpose(qn * wgt)

    @pl.when(i == _NT - 1)
    def _():
        g_ref[...] = gscr[...]
        s_ref[...] = sscr[...]


_CHK = 2048
_HALF = _N // _NC
_NCHK = _HALF // _CHK
_GPC = _CHK // 16


def _sc_scatter_kernel(wqt_hbm, gi2_hbm, out_hbm, acc, acc2, idx_b, dat_b):
    cid = lax.axis_index("c")
    sid = lax.axis_index("s")
    wid = cid * _NS + sid
    half = wid // 16
    c0 = (wid % 16) * 16
    r0 = half * _HALF

    def zbody(k, carry):
        for c in range(16):
            acc[pl.ds(c * _M + k * 16, 16)] = jnp.zeros((16,), jnp.float32)
        return carry
    lax.fori_loop(0, _M // 16, zbody, 0)

    pltpu.sync_copy(gi2_hbm.at[pl.ds(half * (_HALF // _T), _HALF // _T)],
                    idx_b)

    def cbody(ci, carry):
        pltpu.sync_copy(
            wqt_hbm.at[pl.ds(c0, 16), pl.ds(r0 + ci * _CHK, _CHK)], dat_b)

        def gbody(a, c2):
            for b in range(16):
                idx16 = idx_b[ci * (_CHK // _T) + a, pl.ds(b * 16, 16)]
                for c in range(16):
                    d16 = dat_b[c, pl.ds(a * _T + b * 16, 16)]
                    plsc.addupdate_scatter(acc, [idx16 + c * _M], d16)
            return c2
        lax.fori_loop(0, _CHK // _T, gbody, 0)
        return carry
    lax.fori_loop(0, _NCHK, cbody, 0)

    def ubody(k, carry):
        for c in range(16):
            acc2[c, pl.ds(k * 16, 16)] = acc[pl.ds(c * _M + k * 16, 16)]
        return carry
    lax.fori_loop(0, _M // 16, ubody, 0)

    pltpu.sync_copy(acc2, out_hbm.at[half, pl.ds(c0, 16), pl.ds(0, _M)])


def _finish_kernel(parts_ref, keys_ref, upd_ref):
    keys = keys_ref[...]
    st = parts_ref[0] + parts_ref[1]
    upd = jnp.transpose(st) + keys
    n2 = jnp.sum(upd * upd, axis=1, keepdims=True)
    upd_ref[...] = upd * lax.rsqrt(jnp.maximum(n2, 1e-24))


def kernel(query, keys):
    qf = jnp.transpose(query, (0, 2, 3, 1)).reshape(_N, _D)
    f32 = jnp.float32

    cs, cme, kn2 = pl.pallas_call(
        _stats_kernel,
        grid=(_NT,),
        in_specs=[
            pl.BlockSpec((_T, _D), lambda i: (i, 0)),
            pl.BlockSpec((_M, _D), lambda i: (0, 0)),
        ],
        out_specs=[
            pl.BlockSpec((1, _M), lambda i: (0, 0)),
            pl.BlockSpec((1, _M), lambda i: (0, 0)),
            pl.BlockSpec((1, _M), lambda i: (0, 0)),
        ],
        out_shape=[
            jax.ShapeDtypeStruct((1, _M), f32),
            jax.ShapeDtypeStruct((1, _M), f32),
            jax.ShapeDtypeStruct((1, _M), f32),
        ],
    )(qf, keys)

    sm, sq, qcat, g_loss, s_loss, gi, wq = pl.pallas_call(
        _main_kernel,
        grid=(_NT,),
        in_specs=[
            pl.BlockSpec((_T, _D), lambda i: (i, 0)),
            pl.BlockSpec((_M, _D), lambda i: (0, 0)),
            pl.BlockSpec((1, _M), lambda i: (0, 0)),
            pl.BlockSpec((1, _M), lambda i: (0, 0)),
            pl.BlockSpec((1, _M), lambda i: (0, 0)),
        ],
        out_specs=[
            pl.BlockSpec((_T, _M), lambda i: (i, 0)),
            pl.BlockSpec((_T, _M), lambda i: (i, 0)),
            pl.BlockSpec((_T, 2 * _D), lambda i: (i, 0)),
            pl.BlockSpec((_B, 1), lambda i: (0, 0)),
            pl.BlockSpec((_B, 1), lambda i: (0, 0)),
            pl.BlockSpec((_NT, _T), lambda i: (0, 0)),
            pl.BlockSpec((_D, _T), lambda i: (0, i)),
        ],
        out_shape=[
            jax.ShapeDtypeStruct((_N, _M), f32),
            jax.ShapeDtypeStruct((_N, _M), f32),
            jax.ShapeDtypeStruct((_N, 2 * _D), f32),
            jax.ShapeDtypeStruct((_B, 1), f32),
            jax.ShapeDtypeStruct((_B, 1), f32),
            jax.ShapeDtypeStruct((_NT, _T), jnp.int32),
            jax.ShapeDtypeStruct((_D, _N), f32),
        ],
        scratch_shapes=[
            pltpu.VMEM((_B, 1), f32),
            pltpu.VMEM((_B, 1), f32),
        ],
    )(qf, keys, cs, cme, kn2)

    sc_scatter = pl.kernel(
        _sc_scatter_kernel,
        out_type=jax.ShapeDtypeStruct((2, _D, _M), f32),
        mesh=plsc.VectorSubcoreMesh(core_axis_name="c", subcore_axis_name="s",
                                    num_cores=_NC, num_subcores=_NS),
        compiler_params=pltpu.CompilerParams(needs_layout_passes=False),
        scratch_types=[
            pltpu.VMEM((16 * _M,), f32),
            pltpu.VMEM((16, _M), f32),
            pltpu.VMEM((_HALF // _T, _T), jnp.int32),
            pltpu.VMEM((16, _CHK), f32),
        ],
    )
    parts = sc_scatter(wq, gi)

    upd = pl.pallas_call(
        _finish_kernel,
        in_specs=[
            pl.BlockSpec((2, _D, _M), lambda: (0, 0, 0)),
            pl.BlockSpec((_M, _D), lambda: (0, 0)),
        ],
        out_specs=pl.BlockSpec((_M, _D), lambda: (0, 0)),
        out_shape=jax.ShapeDtypeStruct((_M, _D), f32),
    )(parts, keys)

    uq = qcat.reshape(_B, _H, _W, 2 * _D).transpose(0, 3, 1, 2)
    return (uq, upd, sq, sm, g_loss, s_loss)


# speedup vs baseline: 1.0452x; 1.0452x over previous
import jax
import jax.numpy as jnp
from jax import lax
from jax.experimental import pallas as pl
from jax.experimental.pallas import tpu as pltpu
from jax.experimental.pallas import tpu_sc as plsc

_B, _D, _H, _W = 8, 256, 32, 32
_M = 1024
_HW = _H * _W
_N = _B * _HW
_T = 256
_NT = _N // _T
_TPB = _HW // _T
_NEG = -1e30
_NC, _NS = 2, 16
_NW = _NC * _NS
_RPW = _N // _NW


def _norm_rows(q):
    n2 = jnp.sum(q * q, axis=1, keepdims=True)
    return q * lax.rsqrt(jnp.maximum(n2, 1e-24))


def _score_of(qn, keys):
    return lax.dot_general(qn, keys, (((1,), (1,)), ((), ())),
                           preferred_element_type=jnp.float32)


def _split_hi_lo(x):
    hi = x.astype(jnp.bfloat16).astype(jnp.float32)
    return hi, x - hi


def _stats_kernel(qf_ref, keys_ref, cs_ref, cme_ref, kn2_ref):
    i = pl.program_id(0)
    keys = keys_ref[...]
    escore = jnp.exp(_score_of(_norm_rows(qf_ref[...]), keys))

    @pl.when(i == 0)
    def _():
        cs_ref[...] = jnp.zeros((1, _M), jnp.float32)
        cme_ref[...] = jnp.zeros((1, _M), jnp.float32)
        ones_row = jnp.ones((1, _D), jnp.float32)
        kn2_ref[...] = lax.dot_general(ones_row, keys * keys,
                                       (((1,), (1,)), ((), ())),
                                       preferred_element_type=jnp.float32)

    cs_ref[...] += jnp.sum(escore, axis=0, keepdims=True)
    cme_ref[...] = jnp.maximum(cme_ref[...], jnp.max(escore, axis=0, keepdims=True))


def _main_kernel(qf_ref, keys_ref, cs_ref, cme_ref, kn2_ref,
                 sm_ref, sq_ref, qcat_ref, g_ref, s_ref, gi_ref, wq_ref,
                 gscr, sscr):
    i = pl.program_id(0)
    keys = keys_ref[...]
    qn = _norm_rows(qf_ref[...])
    qcat_ref[:, :_D] = qn
    score = _score_of(qn, keys)

    rmax = jnp.max(score, axis=1, keepdims=True)
    e = jnp.exp(score - rmax)
    rsum = jnp.sum(e, axis=1, keepdims=True)
    sm = e * (1.0 / rsum)
    sm_ref[...] = sm

    u = jnp.exp(rmax)
    v = 1.0 / cs_ref[...]
    sq_ref[...] = e * u * v

    qcat_ref[:, _D:] = lax.dot_general(sm, keys, (((1,), (0,)), ((), ())),
                                       preferred_element_type=jnp.float32)

    b1 = score >= rmax
    oh1 = b1.astype(jnp.float32)
    masked = jnp.where(b1, _NEG, score)
    m2 = jnp.max(masked, axis=1, keepdims=True)
    oh2 = (masked >= m2).astype(jnp.float32)

    cme_hi, cme_lo = _split_hi_lo(cme_ref[...])
    io_hi, io_lo = _split_hi_lo(
        lax.broadcasted_iota(jnp.int32, (1, _M), 1).astype(jnp.float32))
    vrows = jnp.concatenate(
        [cme_hi, cme_lo, kn2_ref[...], io_hi, io_lo], axis=0)
    g1 = lax.dot_general(oh1, vrows, (((1,), (1,)), ((), ())),
                         preferred_element_type=jnp.float32)
    cme_g = g1[:, 0:1] + g1[:, 1:2]
    kn2_g = g1[:, 2:3]
    gi_f = g1[:, 3:4] + g1[:, 4:5]
    gi_row = jnp.transpose(jnp.clip(gi_f + 0.5, 0.0, _M - 1.0))
    gi_ref[pl.ds(i, 1), :] = gi_row.astype(jnp.int32)
    kn2_g2 = lax.dot_general(oh2, kn2_ref[...], (((1,), (1,)), ((), ())),
                             preferred_element_type=jnp.float32)

    @pl.when(i == 0)
    def _():
        gscr[...] = jnp.zeros((_B, 1), jnp.float32)
        sscr[...] = jnp.zeros((_B, 1), jnp.float32)

    boh = lax.broadcasted_iota(jnp.int32, (_B, 1), 0) == (i // _TPB)

    d1sq = 1.0 - 2.0 * rmax + kn2_g
    gscr[...] += jnp.where(boh, jnp.sum(d1sq) / (_HW * _D * 1.0), 0.0)

    d2sq = 1.0 - 2.0 * m2 + kn2_g2
    dp = jnp.sqrt(jnp.maximum(d1sq, 0.0))
    dn = jnp.sqrt(jnp.maximum(d2sq, 0.0))
    s_row = jnp.maximum(dp - dn + 1.0, 0.0)
    sscr[...] += jnp.where(boh, jnp.sum(s_row) / (_HW * 1.0), 0.0)

    wgt = u * (1.0 / cme_g)
    wq_ref[...] = jnp.transpose(qn * wgt)

    @pl.when(i == _NT - 1)
    def _():
        g_ref[...] = gscr[...]
        s_ref[...] = sscr[...]


_CHK = 2048
_HALF = _N // _NC
_NCHK = _HALF // _CHK
_GPC = _CHK // 16


def _sc_scatter_kernel(wqt_hbm, gi2_hbm, out_hbm, acc, acc2, idx_b, dat_b):
    cid = lax.axis_index("c")
    sid = lax.axis_index("s")
    wid = cid * _NS + sid
    half = wid // 16
    c0 = (wid % 16) * 16
    r0 = half * _HALF

    def zbody(k, carry):
        for c in range(16):
            acc[pl.ds(c * _M + k * 16, 16)] = jnp.zeros((16,), jnp.float32)
        return carry
    lax.fori_loop(0, _M // 16, zbody, 0)

    pltpu.sync_copy(gi2_hbm.at[pl.ds(half * (_HALF // _T), _HALF // _T)],
                    idx_b)

    def cbody(ci, carry):
        pltpu.sync_copy(
            wqt_hbm.at[pl.ds(c0, 16), pl.ds(r0 + ci * _CHK, _CHK)], dat_b)

        @plsc.parallel_loop(0, _CHK // _T)
        def _gbody(a):
            for b in range(16):
                idx16 = idx_b[ci * (_CHK // _T) + a, pl.ds(b * 16, 16)]
                for c in range(16):
                    d16 = dat_b[c, pl.ds(a * _T + b * 16, 16)]
                    plsc.addupdate_scatter(acc, [idx16 + c * _M], d16)
        return carry
    lax.fori_loop(0, _NCHK, cbody, 0)

    def ubody(k, carry):
        for c in range(16):
            acc2[c, pl.ds(k * 16, 16)] = acc[pl.ds(c * _M + k * 16, 16)]
        return carry
    lax.fori_loop(0, _M // 16, ubody, 0)

    pltpu.sync_copy(acc2, out_hbm.at[half, pl.ds(c0, 16), pl.ds(0, _M)])


def _finish_kernel(parts_ref, keys_ref, upd_ref):
    keys = keys_ref[...]
    st = parts_ref[0] + parts_ref[1]
    upd = jnp.transpose(st) + keys
    n2 = jnp.sum(upd * upd, axis=1, keepdims=True)
    upd_ref[...] = upd * lax.rsqrt(jnp.maximum(n2, 1e-24))


def kernel(query, keys):
    qf = jnp.transpose(query, (0, 2, 3, 1)).reshape(_N, _D)
    f32 = jnp.float32

    cs, cme, kn2 = pl.pallas_call(
        _stats_kernel,
        grid=(_NT,),
        in_specs=[
            pl.BlockSpec((_T, _D), lambda i: (i, 0)),
            pl.BlockSpec((_M, _D), lambda i: (0, 0)),
        ],
        out_specs=[
            pl.BlockSpec((1, _M), lambda i: (0, 0)),
            pl.BlockSpec((1, _M), lambda i: (0, 0)),
            pl.BlockSpec((1, _M), lambda i: (0, 0)),
        ],
        out_shape=[
            jax.ShapeDtypeStruct((1, _M), f32),
            jax.ShapeDtypeStruct((1, _M), f32),
            jax.ShapeDtypeStruct((1, _M), f32),
        ],
    )(qf, keys)

    sm, sq, qcat, g_loss, s_loss, gi, wq = pl.pallas_call(
        _main_kernel,
        grid=(_NT,),
        in_specs=[
            pl.BlockSpec((_T, _D), lambda i: (i, 0)),
            pl.BlockSpec((_M, _D), lambda i: (0, 0)),
            pl.BlockSpec((1, _M), lambda i: (0, 0)),
            pl.BlockSpec((1, _M), lambda i: (0, 0)),
            pl.BlockSpec((1, _M), lambda i: (0, 0)),
        ],
        out_specs=[
            pl.BlockSpec((_T, _M), lambda i: (i, 0)),
            pl.BlockSpec((_T, _M), lambda i: (i, 0)),
            pl.BlockSpec((_T, 2 * _D), lambda i: (i, 0)),
            pl.BlockSpec((_B, 1), lambda i: (0, 0)),
            pl.BlockSpec((_B, 1), lambda i: (0, 0)),
            pl.BlockSpec((_NT, _T), lambda i: (0, 0)),
            pl.BlockSpec((_D, _T), lambda i: (0, i)),
        ],
        out_shape=[
            jax.ShapeDtypeStruct((_N, _M), f32),
            jax.ShapeDtypeStruct((_N, _M), f32),
            jax.ShapeDtypeStruct((_N, 2 * _D), f32),
            jax.ShapeDtypeStruct((_B, 1), f32),
            jax.ShapeDtypeStruct((_B, 1), f32),
            jax.ShapeDtypeStruct((_NT, _T), jnp.int32),
            jax.ShapeDtypeStruct((_D, _N), f32),
        ],
        scratch_shapes=[
            pltpu.VMEM((_B, 1), f32),
            pltpu.VMEM((_B, 1), f32),
        ],
    )(qf, keys, cs, cme, kn2)

    sc_scatter = pl.kernel(
        _sc_scatter_kernel,
        out_type=jax.ShapeDtypeStruct((2, _D, _M), f32),
        mesh=plsc.VectorSubcoreMesh(core_axis_name="c", subcore_axis_name="s",
                                    num_cores=_NC, num_subcores=_NS),
        compiler_params=pltpu.CompilerParams(needs_layout_passes=False),
        scratch_types=[
            pltpu.VMEM((16 * _M,), f32),
            pltpu.VMEM((16, _M), f32),
            pltpu.VMEM((_HALF // _T, _T), jnp.int32),
            pltpu.VMEM((16, _CHK), f32),
        ],
    )
    parts = sc_scatter(wq, gi)

    upd = pl.pallas_call(
        _finish_kernel,
        in_specs=[
            pl.BlockSpec((2, _D, _M), lambda: (0, 0, 0)),
            pl.BlockSpec((_M, _D), lambda: (0, 0)),
        ],
        out_specs=pl.BlockSpec((_M, _D), lambda: (0, 0)),
        out_shape=jax.ShapeDtypeStruct((_M, _D), f32),
    )(parts, keys)

    uq = qcat.reshape(_B, _H, _W, 2 * _D).transpose(0, 3, 1, 2)
    return (uq, upd, sq, sm, g_loss, s_loss)


# speedup vs baseline: 1.0664x; 1.0204x over previous
import jax
import jax.numpy as jnp
from jax import lax
from jax.experimental import pallas as pl
from jax.experimental.pallas import tpu as pltpu
from jax.experimental.pallas import tpu_sc as plsc

_B, _D, _H, _W = 8, 256, 32, 32
_M = 1024
_HW = _H * _W
_N = _B * _HW
_T = 256
_NT = _N // _T
_TPB = _HW // _T
_NEG = -1e30
_NC, _NS = 2, 16
_NW = _NC * _NS
_RPW = _N // _NW


def _norm_rows(q):
    n2 = jnp.sum(q * q, axis=1, keepdims=True)
    return q * lax.rsqrt(jnp.maximum(n2, 1e-24))


def _score_of(qn, keys):
    return lax.dot_general(qn, keys, (((1,), (1,)), ((), ())),
                           preferred_element_type=jnp.float32)


def _split_hi_lo(x):
    hi = x.astype(jnp.bfloat16).astype(jnp.float32)
    return hi, x - hi


def _stats_kernel(qf_ref, keys_ref, cs_ref, cme_ref, kn2_ref):
    i = pl.program_id(0)
    keys = keys_ref[...]
    escore = jnp.exp(_score_of(_norm_rows(qf_ref[...]), keys))

    @pl.when(i == 0)
    def _():
        cs_ref[...] = jnp.zeros((1, _M), jnp.float32)
        cme_ref[...] = jnp.zeros((1, _M), jnp.float32)
        ones_row = jnp.ones((1, _D), jnp.float32)
        kn2_ref[...] = lax.dot_general(ones_row, keys * keys,
                                       (((1,), (1,)), ((), ())),
                                       preferred_element_type=jnp.float32)

    cs_ref[...] += jnp.sum(escore, axis=0, keepdims=True)
    cme_ref[...] = jnp.maximum(cme_ref[...], jnp.max(escore, axis=0, keepdims=True))


def _main_kernel(qf_ref, keys_ref, cs_ref, cme_ref, kn2_ref,
                 sm_ref, sq_ref, qcat_ref, g_ref, s_ref, gi_ref, wq_ref,
                 gscr, sscr):
    i = pl.program_id(0)
    keys = keys_ref[...]
    qn = _norm_rows(qf_ref[...])
    qcat_ref[:, :_D] = qn
    score = _score_of(qn, keys)

    rmax = jnp.max(score, axis=1, keepdims=True)
    e = jnp.exp(score - rmax)
    rsum = jnp.sum(e, axis=1, keepdims=True)
    sm = e * (1.0 / rsum)
    sm_ref[...] = sm

    u = jnp.exp(rmax)
    v = 1.0 / cs_ref[...]
    sq_ref[...] = e * u * v

    qcat_ref[:, _D:] = lax.dot_general(sm, keys, (((1,), (0,)), ((), ())),
                                       preferred_element_type=jnp.float32)

    b1 = score >= rmax
    oh1 = b1.astype(jnp.float32)
    masked = jnp.where(b1, _NEG, score)
    m2 = jnp.max(masked, axis=1, keepdims=True)
    oh2 = (masked >= m2).astype(jnp.float32)

    cme_hi, cme_lo = _split_hi_lo(cme_ref[...])
    io_hi, io_lo = _split_hi_lo(
        lax.broadcasted_iota(jnp.int32, (1, _M), 1).astype(jnp.float32))
    vrows = jnp.concatenate(
        [cme_hi, cme_lo, kn2_ref[...], io_hi, io_lo], axis=0)
    g1 = lax.dot_general(oh1, vrows, (((1,), (1,)), ((), ())),
                         preferred_element_type=jnp.float32)
    cme_g = g1[:, 0:1] + g1[:, 1:2]
    kn2_g = g1[:, 2:3]
    gi_f = g1[:, 3:4] + g1[:, 4:5]
    gi_row = jnp.transpose(jnp.clip(gi_f + 0.5, 0.0, _M - 1.0))
    gi_ref[pl.ds(i, 1), :] = gi_row.astype(jnp.int32)
    kn2_g2 = lax.dot_general(oh2, kn2_ref[...], (((1,), (1,)), ((), ())),
                             preferred_element_type=jnp.float32)

    @pl.when(i == 0)
    def _():
        gscr[...] = jnp.zeros((_B, 1), jnp.float32)
        sscr[...] = jnp.zeros((_B, 1), jnp.float32)

    boh = lax.broadcasted_iota(jnp.int32, (_B, 1), 0) == (i // _TPB)

    d1sq = 1.0 - 2.0 * rmax + kn2_g
    gscr[...] += jnp.where(boh, jnp.sum(d1sq) / (_HW * _D * 1.0), 0.0)

    d2sq = 1.0 - 2.0 * m2 + kn2_g2
    dp = jnp.sqrt(jnp.maximum(d1sq, 0.0))
    dn = jnp.sqrt(jnp.maximum(d2sq, 0.0))
    s_row = jnp.maximum(dp - dn + 1.0, 0.0)
    sscr[...] += jnp.where(boh, jnp.sum(s_row) / (_HW * 1.0), 0.0)

    wgt = u * (1.0 / cme_g)
    wq_ref[...] = jnp.transpose(qn * wgt)

    @pl.when(i == _NT - 1)
    def _():
        g_ref[...] = gscr[...]
        s_ref[...] = sscr[...]


_CHK = 2048
_HALF = _N // _NC
_NCHK = _HALF // _CHK
_GPC = _CHK // 16


def _sc_scatter_kernel(wqt_hbm, gi2_hbm, out_hbm, acc, acc2, idx_b, dat_b):
    cid = lax.axis_index("c")
    sid = lax.axis_index("s")
    wid = cid * _NS + sid
    half = wid // 16
    c0 = (wid % 16) * 16
    r0 = half * _HALF

    @plsc.parallel_loop(0, _M // 16)
    def _zbody(k):
        for c in range(16):
            acc[pl.ds(c * _M + k * 16, 16)] = jnp.zeros((16,), jnp.float32)

    pltpu.sync_copy(gi2_hbm.at[pl.ds(half * (_HALF // _T), _HALF // _T)],
                    idx_b)

    def cbody(ci, carry):
        pltpu.sync_copy(
            wqt_hbm.at[pl.ds(c0, 16), pl.ds(r0 + ci * _CHK, _CHK)], dat_b)

        @plsc.parallel_loop(0, _CHK // _T)
        def _gbody(a):
            for b in range(16):
                idx16 = idx_b[ci * (_CHK // _T) + a, pl.ds(b * 16, 16)]
                for c in range(16):
                    d16 = dat_b[c, pl.ds(a * _T + b * 16, 16)]
                    plsc.addupdate_scatter(acc, [idx16 + c * _M], d16)
        return carry
    lax.fori_loop(0, _NCHK, cbody, 0)

    @plsc.parallel_loop(0, _M // 16)
    def _ubody(k):
        for c in range(16):
            acc2[c, pl.ds(k * 16, 16)] = acc[pl.ds(c * _M + k * 16, 16)]

    pltpu.sync_copy(acc2, out_hbm.at[half, pl.ds(c0, 16), pl.ds(0, _M)])


def _finish_kernel(parts_ref, keys_ref, upd_ref):
    keys = keys_ref[...]
    st = parts_ref[0] + parts_ref[1]
    upd = jnp.transpose(st) + keys
    n2 = jnp.sum(upd * upd, axis=1, keepdims=True)
    upd_ref[...] = upd * lax.rsqrt(jnp.maximum(n2, 1e-24))


def kernel(query, keys):
    qf = jnp.transpose(query, (0, 2, 3, 1)).reshape(_N, _D)
    f32 = jnp.float32

    cs, cme, kn2 = pl.pallas_call(
        _stats_kernel,
        grid=(_NT,),
        in_specs=[
            pl.BlockSpec((_T, _D), lambda i: (i, 0)),
            pl.BlockSpec((_M, _D), lambda i: (0, 0)),
        ],
        out_specs=[
            pl.BlockSpec((1, _M), lambda i: (0, 0)),
            pl.BlockSpec((1, _M), lambda i: (0, 0)),
            pl.BlockSpec((1, _M), lambda i: (0, 0)),
        ],
        out_shape=[
            jax.ShapeDtypeStruct((1, _M), f32),
            jax.ShapeDtypeStruct((1, _M), f32),
            jax.ShapeDtypeStruct((1, _M), f32),
        ],
    )(qf, keys)

    sm, sq, qcat, g_loss, s_loss, gi, wq = pl.pallas_call(
        _main_kernel,
        grid=(_NT,),
        in_specs=[
            pl.BlockSpec((_T, _D), lambda i: (i, 0)),
            pl.BlockSpec((_M, _D), lambda i: (0, 0)),
            pl.BlockSpec((1, _M), lambda i: (0, 0)),
            pl.BlockSpec((1, _M), lambda i: (0, 0)),
            pl.BlockSpec((1, _M), lambda i: (0, 0)),
        ],
        out_specs=[
            pl.BlockSpec((_T, _M), lambda i: (i, 0)),
            pl.BlockSpec((_T, _M), lambda i: (i, 0)),
            pl.BlockSpec((_T, 2 * _D), lambda i: (i, 0)),
            pl.BlockSpec((_B, 1), lambda i: (0, 0)),
            pl.BlockSpec((_B, 1), lambda i: (0, 0)),
            pl.BlockSpec((_NT, _T), lambda i: (0, 0)),
            pl.BlockSpec((_D, _T), lambda i: (0, i)),
        ],
        out_shape=[
            jax.ShapeDtypeStruct((_N, _M), f32),
            jax.ShapeDtypeStruct((_N, _M), f32),
            jax.ShapeDtypeStruct((_N, 2 * _D), f32),
            jax.ShapeDtypeStruct((_B, 1), f32),
            jax.ShapeDtypeStruct((_B, 1), f32),
            jax.ShapeDtypeStruct((_NT, _T), jnp.int32),
            jax.ShapeDtypeStruct((_D, _N), f32),
        ],
        scratch_shapes=[
            pltpu.VMEM((_B, 1), f32),
            pltpu.VMEM((_B, 1), f32),
        ],
    )(qf, keys, cs, cme, kn2)

    sc_scatter = pl.kernel(
        _sc_scatter_kernel,
        out_type=jax.ShapeDtypeStruct((2, _D, _M), f32),
        mesh=plsc.VectorSubcoreMesh(core_axis_name="c", subcore_axis_name="s",
                                    num_cores=_NC, num_subcores=_NS),
        compiler_params=pltpu.CompilerParams(needs_layout_passes=False),
        scratch_types=[
            pltpu.VMEM((16 * _M,), f32),
            pltpu.VMEM((16, _M), f32),
            pltpu.VMEM((_HALF // _T, _T), jnp.int32),
            pltpu.VMEM((16, _CHK), f32),
        ],
    )
    parts = sc_scatter(wq, gi)

    upd = pl.pallas_call(
        _finish_kernel,
        in_specs=[
            pl.BlockSpec((2, _D, _M), lambda: (0, 0, 0)),
            pl.BlockSpec((_M, _D), lambda: (0, 0)),
        ],
        out_specs=pl.BlockSpec((_M, _D), lambda: (0, 0)),
        out_shape=jax.ShapeDtypeStruct((_M, _D), f32),
    )(parts, keys)

    uq = qcat.reshape(_B, _H, _W, 2 * _D).transpose(0, 3, 1, 2)
    return (uq, upd, sq, sm, g_loss, s_loss)
